# baseline (speedup 1.0000x reference)
import functools

import jax
import jax.numpy as jnp
from jax import lax
from jax.experimental import pallas as pl
from jax.experimental.pallas import tpu as pltpu
from jax.experimental.pallas import tpu_sc as plsc

_B, _T, _H, _W, _C = 8, 16, 48, 48, 256
_ROW = _W * _C
_NROW = _B * _T * _H
_P = _T * _H
_NW = 32
_PPW = _P // _NW
_RPW = _PPW * _B
_NBUF = 8
_LA = 6
_LANE = 16
_CC = _C // _LANE


def _sc_body(x_hbm, t_hbm, h_hbm, w_hbm, out_hbm,
             we_v, th_v, r_v, xb_v, sem_in, sem_out):
    cid = lax.axis_index("c")
    sid = lax.axis_index("s")
    wid = sid * 2 + cid
    base_p = wid * _PPW

    pltpu.sync_copy(w_hbm, we_v)
    pltpu.sync_copy(t_hbm, xb_v.at[pl.ds(0, _T * _C)])
    pltpu.sync_copy(h_hbm, xb_v.at[pl.ds(_ROW, _H * _C)])

    def th_pair(pi, _):
        p = base_p + pi
        t = p // _H
        h = lax.rem(p, _H)

        @plsc.parallel_loop(0, _CC, unroll=4)
        def _(c):
            th_v[pl.ds(pi * _C + c * _LANE, _LANE)] = (
                xb_v[pl.ds(t * _C + c * _LANE, _LANE)]
                + xb_v[pl.ds(_ROW + h * _C + c * _LANE, _LANE)])
        return 0
    lax.fori_loop(0, _PPW, th_pair, 0)

    def hbm_row(i):
        p = base_p + i // _B
        b = lax.rem(i, _B)
        return b * _P + p

    def in_desc(i, k):
        r = hbm_row(i)
        return pltpu.make_async_copy(
            x_hbm.at[pl.ds(r * _ROW, _ROW)],
            xb_v.at[pl.ds(k * _ROW, _ROW)],
            sem_in.at[k])

    def out_desc(i, k):
        r = hbm_row(i)
        return pltpu.make_async_copy(
            xb_v.at[pl.ds(k * _ROW, _ROW)],
            out_hbm.at[pl.ds(r * _ROW, _ROW)],
            sem_out.at[k])

    for k in range(_LA):
        in_desc(k, k).start()

    def build_r(pi):
        @plsc.parallel_loop(0, _ROW // _LANE, unroll=8)
        def _(q):
            off = q * _LANE
            coff = pi * _C + lax.rem(q, _CC) * _LANE
            r_v[pl.ds(off, _LANE)] = (
                we_v[pl.ds(off, _LANE)] + th_v[pl.ds(coff, _LANE)])

    def step(g, _):
        for k in range(_NBUF):
            i = g * _NBUF + k

            @pl.when(lax.rem(i, _B) == 0)
            def _():
                build_r(i // _B)

            in_desc(i, k).wait()

            j = i + _LA
            kj = (k + _LA) % _NBUF

            @pl.when(j < _RPW)
            def _():
                @pl.when(j >= _NBUF)
                def _():
                    out_desc(j - _NBUF, kj).wait()
                in_desc(j, kj).start()

            base = k * _ROW

            @plsc.parallel_loop(0, _ROW // _LANE, unroll=8)
            def _(q):
                roff = q * _LANE
                off = base + roff
                xb_v[pl.ds(off, _LANE)] = (
                    xb_v[pl.ds(off, _LANE)] + r_v[pl.ds(roff, _LANE)])

            out_desc(i, k).start()
        return 0

    lax.fori_loop(0, _RPW // _NBUF, step, 0)

    for k in range(_NBUF):
        out_desc(_RPW - _NBUF + k, k).wait()


@jax.jit
def _sc_call(x_flat, t_flat, h_flat, w_flat):
    mesh = plsc.VectorSubcoreMesh(
        core_axis_name="c", subcore_axis_name="s",
        num_cores=2, num_subcores=16)
    fn = pl.kernel(
        _sc_body,
        out_type=jax.ShapeDtypeStruct((_NROW * _ROW,), jnp.float32),
        mesh=mesh,
        scratch_types=[
            pltpu.VMEM((_ROW,), jnp.float32),
            pltpu.VMEM((_PPW * _C,), jnp.float32),
            pltpu.VMEM((_ROW,), jnp.float32),
            pltpu.VMEM((_NBUF * _ROW,), jnp.float32),
            pltpu.SemaphoreType.DMA((_NBUF,)),
            pltpu.SemaphoreType.DMA((_NBUF,)),
        ],
    )
    return fn(x_flat, t_flat, h_flat, w_flat)


def kernel(x, T_embed, H_embed, W_embed):
    B, T, H, W, C = x.shape
    x_flat = x.reshape(-1)
    t_flat = T_embed[:T].reshape(-1)
    h_flat = H_embed[:H].reshape(-1)
    w_flat = W_embed[:W].reshape(-1)
    out_flat = _sc_call(x_flat, t_flat, h_flat, w_flat)
    return out_flat.reshape(x.shape)


# speedup vs baseline: 3.1222x; 3.1222x over previous
import functools

import jax
import jax.numpy as jnp
from jax import lax
from jax.experimental import pallas as pl
from jax.experimental.pallas import tpu as pltpu
from jax.experimental.pallas import tpu_sc as plsc

_B, _T, _H, _W, _C = 8, 16, 48, 48, 256
_ROW = _W * _C
_P = _T * _H
_NW = 32
_PPW = _P // _NW
_RPW = _PPW * _B
_NBUF = 8
_LA = 6
_LANE = 16
_CC = _C // _LANE


def _sc_body(x_hbm, t_hbm, h_hbm, w_hbm, out_hbm,
             we_v, th_v, r_v, xb_v, sem_in, sem_out):
    cid = lax.axis_index("c")
    sid = lax.axis_index("s")
    wid = sid * 2 + cid
    base_p = wid * _PPW

    pltpu.sync_copy(w_hbm, we_v)
    pltpu.sync_copy(t_hbm, xb_v.at[0, pl.ds(0, _T)])
    pltpu.sync_copy(h_hbm, xb_v.at[1])

    def th_pair(pi, _):
        p = base_p + pi
        t = p // _H
        h = lax.rem(p, _H)

        @plsc.parallel_loop(0, _CC, unroll=4)
        def _(c):
            th_v[pi, pl.ds(c * _LANE, _LANE)] = (
                xb_v[0, t, pl.ds(c * _LANE, _LANE)]
                + xb_v[1, h, pl.ds(c * _LANE, _LANE)])
        return 0
    lax.fori_loop(0, _PPW, th_pair, 0)

    def bth(i):
        p = base_p + i // _B
        b = lax.rem(i, _B)
        return b, p // _H, lax.rem(p, _H)

    def in_desc(i, k):
        b, t, h = bth(i)
        return pltpu.make_async_copy(
            x_hbm.at[b, t, h], xb_v.at[k], sem_in.at[k])

    def out_desc(i, k):
        b, t, h = bth(i)
        return pltpu.make_async_copy(
            xb_v.at[k], out_hbm.at[b, t, h], sem_out.at[k])

    for k in range(_LA):
        in_desc(k, k).start()

    def build_r(pi):
        @plsc.parallel_loop(0, _ROW // _LANE, unroll=8)
        def _(q):
            w = q // _CC
            c = lax.rem(q, _CC) * _LANE
            r_v[w, pl.ds(c, _LANE)] = (
                we_v[w, pl.ds(c, _LANE)] + th_v[pi, pl.ds(c, _LANE)])

    def step(g, _):
        for k in range(_NBUF):
            i = g * _NBUF + k

            @pl.when(lax.rem(i, _B) == 0)
            def _():
                build_r(i // _B)

            j = i + _LA
            kj = (k + _LA) % _NBUF

            @pl.when(j < _RPW)
            def _():
                @pl.when(j >= _NBUF)
                def _():
                    out_desc(j - _NBUF, kj).wait()
                in_desc(j, kj).start()

            in_desc(i, k).wait()

            @plsc.parallel_loop(0, _ROW // _LANE, unroll=8)
            def _(q):
                w = q // _CC
                c = lax.rem(q, _CC) * _LANE
                plsc.addupdate(xb_v.at[k, w, pl.ds(c, _LANE)],
                               r_v[w, pl.ds(c, _LANE)])

            out_desc(i, k).start()
        return 0

    lax.fori_loop(0, _RPW // _NBUF, step, 0)

    for k in range(_NBUF):
        out_desc(_RPW - _NBUF + k, k).wait()


@jax.jit
def _sc_call(x, t_emb, h_emb, w_emb):
    mesh = plsc.VectorSubcoreMesh(
        core_axis_name="c", subcore_axis_name="s",
        num_cores=2, num_subcores=16)
    fn = pl.kernel(
        _sc_body,
        out_type=jax.ShapeDtypeStruct((_B, _T, _H, _W, _C), jnp.float32),
        mesh=mesh,
        scratch_types=[
            pltpu.VMEM((_W, _C), jnp.float32),
            pltpu.VMEM((_PPW, _C), jnp.float32),
            pltpu.VMEM((_W, _C), jnp.float32),
            pltpu.VMEM((_NBUF, _W, _C), jnp.float32),
            pltpu.SemaphoreType.DMA((_NBUF,)),
            pltpu.SemaphoreType.DMA((_NBUF,)),
        ],
    )
    return fn(x, t_emb, h_emb, w_emb)


def kernel(x, T_embed, H_embed, W_embed):
    B, T, H, W, C = x.shape
    return _sc_call(x, T_embed[:T], H_embed[:H], W_embed[:W])


# speedup vs baseline: 3.1287x; 1.0021x over previous
import jax
import jax.numpy as jnp
from jax import lax
from jax.experimental import pallas as pl
from jax.experimental.pallas import tpu as pltpu
from jax.experimental.pallas import tpu_sc as plsc

_B, _T, _H, _W, _C = 8, 16, 48, 48, 256
_ROW = _W * _C
_P = _T * _H
_NW = 32
_PPW = _P // _NW
_RPW = _PPW * _B
_NBUF = 8
_LA = 6
_LANE = 16
_CC = _C // _LANE


def _sc_body(x_hbm, t_hbm, h_hbm, w_hbm, out_hbm,
             we_v, th_v, r_v, xb_v, sem_in, sem_out):
    cid = lax.axis_index("c")
    sid = lax.axis_index("s")
    wid = sid * 2 + cid
    base_p = wid * _PPW

    pltpu.sync_copy(w_hbm, we_v)
    pltpu.sync_copy(t_hbm, xb_v.at[0, pl.ds(0, _T)])
    pltpu.sync_copy(h_hbm, xb_v.at[1])

    def th_pair(pi, _):
        p = base_p + pi
        t = p // _H
        h = lax.rem(p, _H)

        @plsc.parallel_loop(0, _CC, unroll=4)
        def _(c):
            th_v[pi, pl.ds(c * _LANE, _LANE)] = (
                xb_v[0, t, pl.ds(c * _LANE, _LANE)]
                + xb_v[1, h, pl.ds(c * _LANE, _LANE)])
        return 0
    lax.fori_loop(0, _PPW, th_pair, 0)

    def bth(i):
        p = base_p + i // _B
        b = lax.rem(i, _B)
        return b, p // _H, lax.rem(p, _H)

    def in_desc(i, k):
        b, t, h = bth(i)
        return pltpu.make_async_copy(
            x_hbm.at[b, t, h], xb_v.at[k], sem_in.at[k])

    def out_desc(i, k):
        b, t, h = bth(i)
        return pltpu.make_async_copy(
            xb_v.at[k], out_hbm.at[b, t, h], sem_out.at[k])

    for k in range(_LA):
        in_desc(k, k).start()

    def build_r(pi):
        @plsc.parallel_loop(0, _ROW // _LANE, unroll=8)
        def _(q):
            w = q // _CC
            c = lax.rem(q, _CC) * _LANE
            r_v[w, pl.ds(c, _LANE)] = (
                we_v[w, pl.ds(c, _LANE)] + th_v[pi, pl.ds(c, _LANE)])

    def step(g, _):
        for k in range(_NBUF):
            i = g * _NBUF + k

            @pl.when(lax.rem(i, _B) == 0)
            def _():
                build_r(i // _B)

            j = i + _LA
            kj = (k + _LA) % _NBUF

            @pl.when(j < _RPW)
            def _():
                @pl.when(j >= _NBUF)
                def _():
                    out_desc(j - _NBUF, kj).wait()
                in_desc(j, kj).start()

            in_desc(i, k).wait()

            @plsc.parallel_loop(0, _ROW // _LANE, unroll=8)
            def _(q):
                w = q // _CC
                c = lax.rem(q, _CC) * _LANE
                plsc.addupdate(xb_v.at[k, w, pl.ds(c, _LANE)],
                               r_v[w, pl.ds(c, _LANE)])

            out_desc(i, k).start()
        return 0

    lax.fori_loop(0, _RPW // _NBUF, step, 0)

    for k in range(_NBUF):
        out_desc(_RPW - _NBUF + k, k).wait()


@jax.jit
def _sc_call(x, t_emb, h_emb, w_emb):
    mesh = plsc.VectorSubcoreMesh(
        core_axis_name="c", subcore_axis_name="s",
        num_cores=2, num_subcores=16)
    fn = pl.kernel(
        _sc_body,
        out_type=jax.ShapeDtypeStruct((_B, _T, _H, _W, _C), jnp.float32),
        mesh=mesh,
        scratch_types=[
            pltpu.VMEM((_W, _C), jnp.float32),
            pltpu.VMEM((_PPW, _C), jnp.float32),
            pltpu.VMEM((_W, _C), jnp.float32),
            pltpu.VMEM((_NBUF, _W, _C), jnp.float32),
            pltpu.SemaphoreType.DMA((_NBUF,)),
            pltpu.SemaphoreType.DMA((_NBUF,)),
        ],
    )
    return fn(x, t_emb, h_emb, w_emb)


def kernel(x, T_embed, H_embed, W_embed):
    B, T, H, W, C = x.shape
    return _sc_call(x, T_embed[:T], H_embed[:H], W_embed[:W])
